# baseline (speedup 1.0000x reference)
import functools

import jax
import jax.numpy as jnp
from jax.experimental import pallas as pl
from jax.experimental.pallas import tpu as pltpu


def _gcn_body(x_ref, graph_ref, w_ref, b_ref, out_ref, *, chunk):
    g = graph_ref[...].astype(jnp.float32)
    L = g.shape[0]
    deg = jnp.sum(g, axis=0) + 1.0
    dinv = jax.lax.rsqrt(deg)
    s = ((g.T + jnp.eye(L, dtype=jnp.float32)) * (
        dinv[:, None] * dinv[None, :])).astype(jnp.bfloat16)

    xb = x_ref[...]
    f = xb.shape[-1]
    xw = jnp.dot(xb.reshape(chunk * L, f).astype(jnp.bfloat16),
                 w_ref[...].astype(jnp.bfloat16),
                 preferred_element_type=jnp.float32).reshape(chunk, L, f)
    sb = jnp.broadcast_to(s, (chunk, L, L))
    agg = jax.lax.dot_general(sb, xw.astype(jnp.bfloat16),
                              (((2,), (1,)), ((0,), (0,))),
                              preferred_element_type=jnp.float32)
    out_ref[...] = jnp.maximum(agg + b_ref[...], 0.0) + xb


def kernel(x, graph, W, b):
    bsz, len_, d = x.shape
    chunk = 32
    grid = (bsz // chunk,)
    out = pl.pallas_call(
        functools.partial(_gcn_body, chunk=chunk),
        grid=grid,
        in_specs=[
            pl.BlockSpec((chunk, len_, d), lambda i: (i, 0, 0)),
            pl.BlockSpec((len_, len_), lambda i: (0, 0)),
            pl.BlockSpec((d, d), lambda i: (0, 0)),
            pl.BlockSpec((1, d), lambda i: (0, 0)),
        ],
        out_specs=pl.BlockSpec((chunk, len_, d), lambda i: (i, 0, 0)),
        out_shape=jax.ShapeDtypeStruct((bsz, len_, d), x.dtype),
        compiler_params=pltpu.CompilerParams(
            dimension_semantics=("parallel",)),
    )(x, graph, W, b.reshape(1, d))
    return out


# speedup vs baseline: 1.2419x; 1.2419x over previous
import functools

import jax
import jax.numpy as jnp
from jax.experimental import pallas as pl
from jax.experimental.pallas import tpu as pltpu


def _gcn_body(x_ref, graph_ref, w_ref, b_ref, out_ref, *, chunk):
    g = graph_ref[...].astype(jnp.float32)
    L = g.shape[0]
    deg = jnp.sum(g, axis=0) + 1.0
    dinv = jax.lax.rsqrt(deg)
    s = ((g.T + jnp.eye(L, dtype=jnp.float32)) * (
        dinv[:, None] * dinv[None, :])).astype(jnp.bfloat16)

    z = jnp.zeros((L, L), jnp.bfloat16)
    s2 = jnp.concatenate(
        [jnp.concatenate([s, z], axis=1), jnp.concatenate([z, s], axis=1)],
        axis=0)

    xb = x_ref[...]
    f = xb.shape[-1]
    xw = jnp.dot(xb.reshape(chunk * L, f).astype(jnp.bfloat16),
                 w_ref[...].astype(jnp.bfloat16),
                 preferred_element_type=jnp.float32)
    xw2 = xw.reshape(chunk // 2, 2 * L, f)
    sb = jnp.broadcast_to(s2, (chunk // 2, 2 * L, 2 * L))
    agg = jax.lax.dot_general(sb, xw2.astype(jnp.bfloat16),
                              (((2,), (1,)), ((0,), (0,))),
                              preferred_element_type=jnp.float32)
    out_ref[...] = jnp.maximum(agg.reshape(chunk, L, f) + b_ref[...],
                               0.0) + xb


def kernel(x, graph, W, b):
    bsz, len_, d = x.shape
    chunk = 64
    grid = (bsz // chunk,)
    out = pl.pallas_call(
        functools.partial(_gcn_body, chunk=chunk),
        grid=grid,
        in_specs=[
            pl.BlockSpec((chunk, len_, d), lambda i: (i, 0, 0)),
            pl.BlockSpec((len_, len_), lambda i: (0, 0)),
            pl.BlockSpec((d, d), lambda i: (0, 0)),
            pl.BlockSpec((1, d), lambda i: (0, 0)),
        ],
        out_specs=pl.BlockSpec((chunk, len_, d), lambda i: (i, 0, 0)),
        out_shape=jax.ShapeDtypeStruct((bsz, len_, d), x.dtype),
        compiler_params=pltpu.CompilerParams(
            dimension_semantics=("parallel",)),
    )(x, graph, W, b.reshape(1, d))
    return out
